# baseline (speedup 1.0000x reference)
import functools

import jax
import jax.numpy as jnp
from jax import lax
from jax.experimental import pallas as pl
from jax.experimental.pallas import tpu as pltpu
from jax.experimental.pallas import tpu_sc as plsc

N_NODES = 10000
N_EDGES = 320000
D = 128

N_PAD = 10240
SLAB = 16
NSLAB = D // SLAB

NC = 2
NS = 16
NW = NC * NS
K = 125
NROWS = N_EDGES // K
ROWS_COL = NROWS // NS
ROWS_DEG = NROWS // NW
NQ = 8
ROWS_PER_TILE = N_PAD // NS

_sc_mesh = plsc.VectorSubcoreMesh(core_axis_name="c", subcore_axis_name="s")


@functools.partial(
    pl.kernel,
    out_type=(
        jax.ShapeDtypeStruct((NSLAB, N_PAD, SLAB), jnp.float32),
        jax.ShapeDtypeStruct((NC, N_PAD, SLAB), jnp.float32),
    ),
    mesh=_sc_mesh,
    compiler_params=pltpu.CompilerParams(use_tc_tiling_on_sc=False),
    scratch_types=(
        pltpu.VMEM((ROWS_COL, K), jnp.int32),
        pltpu.VMEM((ROWS_COL, K), jnp.int32),
        [pltpu.VMEM((K, 32), jnp.float32) for _ in range(NQ)],
        pltpu.VMEM((K, SLAB), jnp.float32),
        pltpu.VMEM((ROWS_PER_TILE, SLAB), jnp.float32),
        pltpu.VMEM_SHARED((N_PAD, SLAB), jnp.float32),
        [pltpu.SemaphoreType.DMA for _ in range(NQ)],
        [pltpu.SemaphoreType.DMA for _ in range(NQ)],
    ),
)
def _sc_aggregate(src_hbm, dst_hbm, xcols_hbm, agg_out, deg_out,
                  src_v, dst_v, slabs, ones_v, zbuf, acc_sh, gsems, ssems):
    c = lax.axis_index("c")
    s = lax.axis_index("s")

    zero16 = jnp.zeros((16,), jnp.float32)
    one16 = jnp.ones((16,), jnp.float32)

    def _fill_o(i, carry):
        ones_v[i] = one16
        return carry
    lax.fori_loop(0, K, _fill_o, 0)

    def _fill_z(i, carry):
        zbuf[i] = zero16
        return carry
    lax.fori_loop(0, ROWS_PER_TILE, _fill_z, 0)

    row0 = s * ROWS_PER_TILE

    def _acc_pass(nrows, chunk_row0, quad_body, out_view):
        pltpu.sync_copy(zbuf, acc_sh.at[pl.ds(row0, ROWS_PER_TILE)])
        if chunk_row0 is not None:
            pltpu.sync_copy(dst_hbm.at[pl.ds(chunk_row0, nrows)],
                            dst_v.at[pl.ds(0, nrows)])
        plsc.subcore_barrier()

        def _quad(q, carry):
            quad_body(q * NQ)
            return carry
        lax.fori_loop(0, nrows // NQ, _quad, 0)
        plsc.subcore_barrier()

        pltpu.sync_copy(acc_sh.at[pl.ds(row0, ROWS_PER_TILE)], out_view)

    pltpu.sync_copy(src_hbm.at[pl.ds(s * ROWS_COL, ROWS_COL)], src_v)
    pltpu.sync_copy(dst_hbm.at[pl.ds(s * ROWS_COL, ROWS_COL)], dst_v)

    for p in range(4):
        pp = p

        def _col_quad(r, _pp=pp):
            gs = [pltpu.async_copy(
                      xcols_hbm.at[_pp].at[src_v.at[r + j]],
                      slabs[j], gsems[j])
                  for j in range(NQ)]
            for j in range(NQ):
                gs[j].wait()

        _acc_pass(ROWS_COL, None, _col_quad,
                  agg_out.at[pp, pl.ds(row0, ROWS_PER_TILE)])

    def _deg_quad(r):
        ss = [pltpu.async_copy(
                  ones_v, acc_sh.at[dst_v.at[r + j]], ssems[j], add=True)
              for j in range(NQ)]
        for j in range(NQ):
            ss[j].wait()

    _acc_pass(ROWS_DEG, (c * NS + s) * ROWS_DEG, _deg_quad,
              deg_out.at[c, pl.ds(row0, ROWS_PER_TILE)])


R_TC = 1000
GRID = N_NODES // R_TC


def _dense_body(x_ref, agg_ref, deg_ref, ws_ref, wnr_ref, b_ref, o_ref):
    d = deg_ref[0, :, 0:1] + deg_ref[1, :, 0:1]
    recip = 1.0 / jnp.maximum(d, 1.0)
    acc = lax.dot_general(x_ref[...], ws_ref[...],
                          (((1,), (1,)), ((), ())),
                          preferred_element_type=jnp.float32)
    for p in range(NSLAB):
        h_p = agg_ref[p] * recip
        acc = acc + lax.dot_general(h_p, wnr_ref[p],
                                    (((1,), (0,)), ((), ())),
                                    preferred_element_type=jnp.float32)
    o_ref[...] = jnp.maximum(acc + b_ref[...], 0.0)


_dense = pl.pallas_call(
    _dense_body,
    grid=(GRID,),
    in_specs=[
        pl.BlockSpec((R_TC, D), lambda i: (i, 0)),
        pl.BlockSpec((NSLAB, R_TC, SLAB), lambda i: (0, i, 0)),
        pl.BlockSpec((NC, R_TC, SLAB), lambda i: (0, i, 0)),
        pl.BlockSpec((D, D), lambda i: (0, 0)),
        pl.BlockSpec((NSLAB, SLAB, D), lambda i: (0, 0, 0)),
        pl.BlockSpec((1, D), lambda i: (0, 0)),
    ],
    out_specs=pl.BlockSpec((R_TC, D), lambda i: (i, 0)),
    out_shape=jax.ShapeDtypeStruct((N_NODES, D), jnp.float32),
)


def kernel(x, edge_index, W_self, W_neigh, b):
    x_cols = x.reshape(N_NODES, 4, 32).transpose(1, 0, 2)
    wn_r = W_neigh.reshape(D, NSLAB, SLAB).transpose(1, 2, 0)
    src2 = edge_index[0].reshape(NROWS, K)
    dst2 = edge_index[1].reshape(NROWS, K)
    agg, deg = _sc_aggregate(src2, dst2, x_cols)
    return _dense(x, agg, deg, W_self, wn_r, b.reshape(1, D))


# speedup vs baseline: 1.0905x; 1.0905x over previous
import functools

import jax
import jax.numpy as jnp
from jax import lax
from jax.experimental import pallas as pl
from jax.experimental.pallas import tpu as pltpu
from jax.experimental.pallas import tpu_sc as plsc

N_NODES = 10000
N_EDGES = 320000
D = 128

N_PAD = 10240
SLAB = 16
NSLAB = D // SLAB

NC = 2
NS = 16
NW = NC * NS
K = 250
NROWS = N_EDGES // K
ROWS_COL = NROWS // NS
ROWS_DEG = NROWS // NW
NQ = 8
ROWS_PER_TILE = N_PAD // NS

_sc_mesh = plsc.VectorSubcoreMesh(core_axis_name="c", subcore_axis_name="s")


@functools.partial(
    pl.kernel,
    out_type=(
        jax.ShapeDtypeStruct((NSLAB, N_PAD, SLAB), jnp.float32),
        jax.ShapeDtypeStruct((NC, N_PAD, SLAB), jnp.float32),
    ),
    mesh=_sc_mesh,
    compiler_params=pltpu.CompilerParams(use_tc_tiling_on_sc=False),
    scratch_types=(
        pltpu.VMEM((ROWS_COL, K), jnp.int32),
        pltpu.VMEM((ROWS_COL, K), jnp.int32),
        [pltpu.VMEM((K, SLAB), jnp.float32) for _ in range(NQ)],
        pltpu.VMEM((K, SLAB), jnp.float32),
        pltpu.VMEM((ROWS_PER_TILE, SLAB), jnp.float32),
        pltpu.VMEM_SHARED((N_PAD, SLAB), jnp.float32),
        [pltpu.SemaphoreType.DMA for _ in range(NQ)],
        [pltpu.SemaphoreType.DMA for _ in range(NQ)],
    ),
)
def _sc_aggregate(src_hbm, dst_hbm, xcols_hbm, agg_out, deg_out,
                  src_v, dst_v, slabs, ones_v, zbuf, acc_sh, gsems, ssems):
    c = lax.axis_index("c")
    s = lax.axis_index("s")

    zero16 = jnp.zeros((16,), jnp.float32)
    one16 = jnp.ones((16,), jnp.float32)

    def _fill_o(i, carry):
        ones_v[i] = one16
        return carry
    lax.fori_loop(0, K, _fill_o, 0)

    def _fill_z(i, carry):
        zbuf[i] = zero16
        return carry
    lax.fori_loop(0, ROWS_PER_TILE, _fill_z, 0)

    row0 = s * ROWS_PER_TILE

    def _acc_pass(nrows, chunk_row0, quad_body, out_view):
        pltpu.sync_copy(zbuf, acc_sh.at[pl.ds(row0, ROWS_PER_TILE)])
        if chunk_row0 is not None:
            pltpu.sync_copy(dst_hbm.at[pl.ds(chunk_row0, nrows)],
                            dst_v.at[pl.ds(0, nrows)])
        plsc.subcore_barrier()

        def _quad(q, carry):
            quad_body(q * NQ)
            return carry
        lax.fori_loop(0, nrows // NQ, _quad, 0)
        plsc.subcore_barrier()

        pltpu.sync_copy(acc_sh.at[pl.ds(row0, ROWS_PER_TILE)], out_view)

    pltpu.sync_copy(src_hbm.at[pl.ds(s * ROWS_COL, ROWS_COL)], src_v)
    pltpu.sync_copy(dst_hbm.at[pl.ds(s * ROWS_COL, ROWS_COL)], dst_v)

    for p in range(NSLAB // NC):
        pp = c * (NSLAB // NC) + p

        def _col_quad(r, _pp=pp):
            gs = [pltpu.async_copy(
                      xcols_hbm.at[_pp].at[src_v.at[r + j]],
                      slabs[j], gsems[j])
                  for j in range(NQ)]
            ss = []
            for j in range(NQ):
                gs[j].wait()
                ss.append(pltpu.async_copy(
                    slabs[j], acc_sh.at[dst_v.at[r + j]], ssems[j],
                    add=True))
            for j in range(NQ):
                ss[j].wait()

        _acc_pass(ROWS_COL, None, _col_quad,
                  agg_out.at[pp, pl.ds(row0, ROWS_PER_TILE)])

    def _deg_quad(r):
        ss = [pltpu.async_copy(
                  ones_v, acc_sh.at[dst_v.at[r + j]], ssems[j], add=True)
              for j in range(NQ)]
        for j in range(NQ):
            ss[j].wait()

    _acc_pass(ROWS_DEG, (c * NS + s) * ROWS_DEG, _deg_quad,
              deg_out.at[c, pl.ds(row0, ROWS_PER_TILE)])


R_TC = 1000
GRID = N_NODES // R_TC


def _dense_body(x_ref, agg_ref, deg_ref, ws_ref, wnr_ref, b_ref, o_ref):
    d = deg_ref[0, :, 0:1] + deg_ref[1, :, 0:1]
    recip = 1.0 / jnp.maximum(d, 1.0)
    acc = lax.dot_general(x_ref[...], ws_ref[...],
                          (((1,), (1,)), ((), ())),
                          preferred_element_type=jnp.float32)
    for p in range(NSLAB):
        h_p = agg_ref[p] * recip
        acc = acc + lax.dot_general(h_p, wnr_ref[p],
                                    (((1,), (0,)), ((), ())),
                                    preferred_element_type=jnp.float32)
    o_ref[...] = jnp.maximum(acc + b_ref[...], 0.0)


_dense = pl.pallas_call(
    _dense_body,
    grid=(GRID,),
    in_specs=[
        pl.BlockSpec((R_TC, D), lambda i: (i, 0)),
        pl.BlockSpec((NSLAB, R_TC, SLAB), lambda i: (0, i, 0)),
        pl.BlockSpec((NC, R_TC, SLAB), lambda i: (0, i, 0)),
        pl.BlockSpec((D, D), lambda i: (0, 0)),
        pl.BlockSpec((NSLAB, SLAB, D), lambda i: (0, 0, 0)),
        pl.BlockSpec((1, D), lambda i: (0, 0)),
    ],
    out_specs=pl.BlockSpec((R_TC, D), lambda i: (i, 0)),
    out_shape=jax.ShapeDtypeStruct((N_NODES, D), jnp.float32),
)


def kernel(x, edge_index, W_self, W_neigh, b):
    x_cols = x.reshape(N_NODES, NSLAB, SLAB).transpose(1, 0, 2)
    wn_r = W_neigh.reshape(D, NSLAB, SLAB).transpose(1, 2, 0)
    src2 = edge_index[0].reshape(NROWS, K)
    dst2 = edge_index[1].reshape(NROWS, K)
    agg, deg = _sc_aggregate(src2, dst2, x_cols)
    return _dense(x, agg, deg, W_self, wn_r, b.reshape(1, D))
